# Initial kernel scaffold; baseline (speedup 1.0000x reference)
import jax
import jax.numpy as jnp
from jax.experimental import pallas as pl


def kernel(x, edge_index, W_e1, b_e1, W_e2, b_e2, W_a1, b_a1, W_a2, b_a2, W_s, b_s):
    raise NotImplementedError("write your pallas kernel here")


# speedup vs baseline: 4.2839x; 4.2839x over previous
import functools

import jax
import jax.numpy as jnp
from jax import lax
from jax.experimental import pallas as pl
from jax.experimental.pallas import tpu as pltpu
from jax.experimental.pallas import tpu_sc as plsc

N = 10000
E = 320000
D_IN = 128
H = 64
C = 128

NC = 2
NS = 16
K = 128
CH = 160
EPT = K * CH
E_PAD = EPT * NS
N_PAD = 10240
HALF = N_PAD // NC
ACCR = HALF + 128
ZPT = ACCR // NS

_mesh = plsc.VectorSubcoreMesh(core_axis_name="c", subcore_axis_name="s")


OPT = HALF // NS


def _remap(dst_v, cc, cid):
    lo = cid * HALF
    for g in range(K // 16):
        iv = dst_v[cc, pl.ds(g * 16, 16)]
        ok = jnp.logical_and(iv >= lo, iv < lo + HALF)
        dst_v[cc, pl.ds(g * 16, 16)] = jnp.where(ok, iv - lo, HALF)


@functools.partial(
    pl.kernel,
    out_type=jax.ShapeDtypeStruct((N_PAD, C), jnp.float32),
    mesh=_mesh,
    scratch_types=[
        pltpu.VMEM((CH, K), jnp.int32),
        pltpu.VMEM((CH, K), jnp.int32),
        pltpu.VMEM((K, C), jnp.float32),
        pltpu.VMEM((K, C), jnp.float32),
        pltpu.VMEM_SHARED((ACCR, C), jnp.float32),
        pltpu.SemaphoreType.DMA,
        pltpu.SemaphoreType.DMA,
    ],
)
def _segsum(h_hbm, srcp_hbm, dstp_hbm, zeros_hbm, out_hbm,
            src_v, dst_v, rows0, rows1, acc, sem0, sem1):
    cid = lax.axis_index("c")
    sid = lax.axis_index("s")
    pltpu.sync_copy(srcp_hbm.at[sid], src_v)
    pltpu.sync_copy(dstp_hbm.at[sid], dst_v)
    pltpu.sync_copy(zeros_hbm, acc.at[pl.ds(sid * ZPT, ZPT)])
    plsc.subcore_barrier()

    bufs = ((rows0, sem0), (rows1, sem1))
    pltpu.async_copy(h_hbm.at[src_v.at[0]], rows0, sem0)
    pltpu.async_copy(h_hbm.at[src_v.at[1]], rows1, sem1)

    def body(i, carry):
        for b, (rows, sem) in enumerate(bufs):
            cc = 2 * i + b
            _remap(dst_v, cc, cid)
            pltpu.make_async_copy(h_hbm.at[src_v.at[cc]], rows, sem).wait()
            pltpu.sync_copy(rows, acc.at[dst_v.at[cc]], add=True)

            @pl.when(cc + 2 < CH)
            def _():
                pltpu.async_copy(h_hbm.at[src_v.at[cc + 2]], rows, sem)

        return carry

    lax.fori_loop(0, CH // 2, body, 0)
    plsc.subcore_barrier()
    pltpu.sync_copy(acc.at[pl.ds(sid * OPT, OPT)],
                    out_hbm.at[pl.ds(cid * HALF + sid * OPT, OPT)])


@functools.partial(
    pl.kernel,
    out_type=jax.ShapeDtypeStruct((N_PAD, C), jnp.float32),
    mesh=_mesh,
    scratch_types=[
        pltpu.VMEM((CH, K), jnp.int32),
        pltpu.VMEM((K, C), jnp.float32),
        pltpu.VMEM_SHARED((ACCR, C), jnp.float32),
    ],
)
def _deg(dstp_hbm, ones_hbm, zeros_hbm, out_hbm, dst_v, ones_v, acc):
    cid = lax.axis_index("c")
    sid = lax.axis_index("s")
    pltpu.sync_copy(dstp_hbm.at[sid], dst_v)
    pltpu.sync_copy(ones_hbm, ones_v)
    pltpu.sync_copy(zeros_hbm, acc.at[pl.ds(sid * ZPT, ZPT)])
    plsc.subcore_barrier()

    def body(c, carry):
        _remap(dst_v, c, cid)
        pltpu.sync_copy(ones_v, acc.at[dst_v.at[c]], add=True)
        return carry

    lax.fori_loop(0, CH, body, 0)
    plsc.subcore_barrier()
    pltpu.sync_copy(acc.at[pl.ds(sid * OPT, OPT)],
                    out_hbm.at[pl.ds(cid * HALF + sid * OPT, OPT)])


BM = 2048


def _finish_deg_body(deg_ref, o_ref):
    o_ref[...] = lax.rsqrt(deg_ref[...][:, 0:1] + 1.0)


_finish_deg = pl.pallas_call(
    _finish_deg_body,
    grid=(N_PAD // BM,),
    in_specs=[pl.BlockSpec((BM, C), lambda i: (i, 0))],
    out_specs=pl.BlockSpec((BM, 1), lambda i: (i, 0)),
    out_shape=jax.ShapeDtypeStruct((N_PAD, 1), jnp.float32),
)


def _mm_scale_body(p_ref, w_ref, dv_ref, o_ref):
    o_ref[...] = dv_ref[...] * jnp.dot(
        p_ref[...], w_ref[...], preferred_element_type=jnp.float32)


_mm_scale = pl.pallas_call(
    _mm_scale_body,
    grid=(N_PAD // BM,),
    in_specs=[
        pl.BlockSpec((BM, C), lambda i: (i, 0)),
        pl.BlockSpec((C, C), lambda i: (0, 0)),
        pl.BlockSpec((BM, 1), lambda i: (i, 0)),
    ],
    out_specs=pl.BlockSpec((BM, C), lambda i: (i, 0)),
    out_shape=jax.ShapeDtypeStruct((N_PAD, C), jnp.float32),
)


def _combine_body(seg_ref, u_ref, dv_ref, b_ref, o_ref, *, crelu):
    t = dv_ref[...] * (seg_ref[...] + u_ref[...]) + b_ref[...]
    if crelu == t.shape[1]:
        t = jnp.maximum(t, 0.0)
    elif crelu > 0:
        cols = lax.broadcasted_iota(jnp.int32, t.shape, 1)
        t = jnp.where(cols < crelu, jnp.maximum(t, 0.0), t)
    o_ref[...] = t


def _make_combine(crelu):
    return pl.pallas_call(
        functools.partial(_combine_body, crelu=crelu),
        grid=(N_PAD // BM,),
        in_specs=[
            pl.BlockSpec((BM, C), lambda i: (i, 0)),
            pl.BlockSpec((BM, C), lambda i: (i, 0)),
            pl.BlockSpec((BM, 1), lambda i: (i, 0)),
            pl.BlockSpec((1, C), lambda i: (0, 0)),
        ],
        out_specs=pl.BlockSpec((BM, C), lambda i: (i, 0)),
        out_shape=jax.ShapeDtypeStruct((N_PAD, C), jnp.float32),
    )


_combine_relu = _make_combine(C)
_combine_plain = _make_combine(0)
_combine_relu64 = _make_combine(64)

BS = 512


def _bigmm_body(a_ref, b_ref, o_ref):
    o_ref[...] = lax.dot_general(
        a_ref[...], b_ref[...], (((1,), (1,)), ((), ())),
        preferred_element_type=jnp.float32)


_bigmm = pl.pallas_call(
    _bigmm_body,
    grid=(pl.cdiv(N, BS), pl.cdiv(N, BS)),
    in_specs=[
        pl.BlockSpec((BS, H), lambda i, j: (i, 0)),
        pl.BlockSpec((BS, H), lambda i, j: (j, 0)),
    ],
    out_specs=pl.BlockSpec((BS, BS), lambda i, j: (i, j)),
    out_shape=jax.ShapeDtypeStruct((N, N), jnp.float32),
)


def _pad_w(w):
    out = jnp.zeros((C, C), jnp.float32)
    return out.at[: w.shape[0], : w.shape[1]].set(w)


def _pad_b(b):
    out = jnp.zeros((C,), jnp.float32)
    return out.at[: b.shape[0]].set(b).reshape(1, C)


def kernel(x, edge_index, W_e1, b_e1, W_e2, b_e2, W_a1, b_a1, W_a2, b_a2,
           W_s, b_s):
    src = edge_index[0].astype(jnp.int32)
    dst = edge_index[1].astype(jnp.int32)
    pad = jnp.full((E_PAD - E,), N_PAD - 1, jnp.int32)
    srcp = jnp.concatenate([src, pad]).reshape(NS, CH, K)
    dstp = jnp.concatenate([dst, pad]).reshape(NS, CH, K)

    xp = jnp.zeros((N_PAD, C), jnp.float32).at[:N].set(x)
    ones128 = jnp.ones((K, C), jnp.float32)
    z128 = jnp.zeros((ZPT, C), jnp.float32)

    deg2 = _deg(dstp, ones128, z128)
    dinv = _finish_deg(deg2)

    u1 = _mm_scale(xp, _pad_w(W_e1), dinv)
    seg1 = _segsum(u1, srcp, dstp, z128)
    h = _combine_relu(seg1, u1, dinv, _pad_b(b_e1))

    u2 = _mm_scale(h, _pad_w(W_e2), dinv)
    seg2 = _segsum(u2, srcp, dstp, z128)
    emb_p = _combine_plain(seg2, u2, dinv, _pad_b(b_e2))

    W3 = _pad_w(jnp.concatenate([W_a1, W_s], axis=1))
    b3 = _pad_b(jnp.concatenate([b_a1, b_s]))
    u3 = _mm_scale(emb_p, W3, dinv)
    seg3 = _segsum(u3, srcp, dstp, z128)
    t3 = _combine_relu64(seg3, u3, dinv, b3)
    hs = t3[:N, H:]

    W4 = _pad_w(jnp.zeros((C, C), jnp.float32).at[:H].set(W_a2))
    u4 = _mm_scale(t3, W4, dinv)
    seg4 = _segsum(u4, srcp, dstp, z128)
    x_p = _combine_plain(seg4, u4, dinv, b_a2.reshape(1, C))

    s_ = _bigmm(hs, hs)
    return (x_p[:N], s_, emb_p[:N, :H])


# speedup vs baseline: 6.7557x; 1.5770x over previous
import functools

import jax
import jax.numpy as jnp
from jax import lax
from jax.experimental import pallas as pl
from jax.experimental.pallas import tpu as pltpu
from jax.experimental.pallas import tpu_sc as plsc

N = 10000
E = 320000
D_IN = 128
H = 64
C = 128

NC = 2
NS = 16
NW = NC * NS
K = 128
CH = 80
EPT = K * CH
E_PAD = EPT * NW
N_PAD = 10240
RPT = N_PAD // NS
IB = 8
NB = CH // IB

_mesh = plsc.VectorSubcoreMesh(core_axis_name="c", subcore_axis_name="s")


@functools.partial(
    pl.kernel,
    out_type=jax.ShapeDtypeStruct((NC, N_PAD, C), jnp.float32),
    mesh=_mesh,
    scratch_types=[
        pltpu.VMEM((IB, K), jnp.int32),
        pltpu.VMEM((IB, K), jnp.int32),
        pltpu.VMEM((K, C), jnp.float32),
        pltpu.VMEM((K, C), jnp.float32),
        pltpu.VMEM_SHARED((N_PAD, C), jnp.float32),
        pltpu.SemaphoreType.DMA,
        pltpu.SemaphoreType.DMA,
    ],
)
def _segsum(h_hbm, srcp_hbm, dstp_hbm, zeros_hbm, out_hbm,
            src_v, dst_v, rows0, rows1, acc, sem0, sem1):
    cid = lax.axis_index("c")
    sid = lax.axis_index("s")
    wid = sid * NC + cid
    r0 = sid * RPT
    pltpu.sync_copy(zeros_hbm, acc.at[pl.ds(r0, RPT)])
    plsc.subcore_barrier()

    bufs = ((rows0, sem0), (rows1, sem1))

    def block(nb, carry):
        pltpu.sync_copy(srcp_hbm.at[wid, pl.ds(nb * IB, IB)], src_v)
        pltpu.sync_copy(dstp_hbm.at[wid, pl.ds(nb * IB, IB)], dst_v)
        pltpu.async_copy(h_hbm.at[src_v.at[0]], rows0, sem0)
        pltpu.async_copy(h_hbm.at[src_v.at[1]], rows1, sem1)
        for j in range(IB):
            rows, sem = bufs[j % 2]
            pltpu.make_async_copy(h_hbm.at[src_v.at[j]], rows, sem).wait()
            pltpu.sync_copy(rows, acc.at[dst_v.at[j]], add=True)
            if j + 2 < IB:
                pltpu.async_copy(h_hbm.at[src_v.at[j + 2]], rows, sem)
        return carry

    lax.fori_loop(0, NB, block, 0)
    plsc.subcore_barrier()
    pltpu.sync_copy(acc.at[pl.ds(r0, RPT)],
                    out_hbm.at[cid, pl.ds(r0, RPT)])


@functools.partial(
    pl.kernel,
    out_type=jax.ShapeDtypeStruct((NC, N_PAD, C), jnp.float32),
    mesh=_mesh,
    scratch_types=[
        pltpu.VMEM((IB, K), jnp.int32),
        pltpu.VMEM((K, C), jnp.float32),
        pltpu.VMEM_SHARED((N_PAD, C), jnp.float32),
    ],
)
def _deg(dstp_hbm, ones_hbm, zeros_hbm, out_hbm, dst_v, ones_v, acc):
    cid = lax.axis_index("c")
    sid = lax.axis_index("s")
    wid = sid * NC + cid
    pltpu.sync_copy(ones_hbm, ones_v)
    r0 = sid * RPT
    pltpu.sync_copy(zeros_hbm, acc.at[pl.ds(r0, RPT)])
    plsc.subcore_barrier()

    def block(nb, carry):
        pltpu.sync_copy(dstp_hbm.at[wid, pl.ds(nb * IB, IB)], dst_v)
        for j in range(IB):
            pltpu.sync_copy(ones_v, acc.at[dst_v.at[j]], add=True)
        return carry

    lax.fori_loop(0, NB, block, 0)
    plsc.subcore_barrier()
    pltpu.sync_copy(acc.at[pl.ds(r0, RPT)], out_hbm.at[cid, pl.ds(r0, RPT)])


BM = 2048


def _finish_deg_body(deg_ref, o_ref):
    d = deg_ref[...]
    o_ref[...] = lax.rsqrt(d[0, :, 0:1] + d[1, :, 0:1] + 1.0)


_finish_deg = pl.pallas_call(
    _finish_deg_body,
    grid=(N_PAD // BM,),
    in_specs=[pl.BlockSpec((NC, BM, C), lambda i: (0, i, 0))],
    out_specs=pl.BlockSpec((BM, 1), lambda i: (i, 0)),
    out_shape=jax.ShapeDtypeStruct((N_PAD, 1), jnp.float32),
)


def _mm_scale_body(p_ref, w_ref, dv_ref, o_ref):
    o_ref[...] = dv_ref[...] * jnp.dot(
        p_ref[...], w_ref[...], preferred_element_type=jnp.float32)


_mm_scale = pl.pallas_call(
    _mm_scale_body,
    grid=(N_PAD // BM,),
    in_specs=[
        pl.BlockSpec((BM, C), lambda i: (i, 0)),
        pl.BlockSpec((C, C), lambda i: (0, 0)),
        pl.BlockSpec((BM, 1), lambda i: (i, 0)),
    ],
    out_specs=pl.BlockSpec((BM, C), lambda i: (i, 0)),
    out_shape=jax.ShapeDtypeStruct((N_PAD, C), jnp.float32),
)


def _combine_body(seg_ref, u_ref, dv_ref, b_ref, o_ref, *, crelu):
    t = dv_ref[...] * (seg_ref[0] + seg_ref[1] + u_ref[...]) + b_ref[...]
    if crelu == t.shape[1]:
        t = jnp.maximum(t, 0.0)
    elif crelu > 0:
        cols = lax.broadcasted_iota(jnp.int32, t.shape, 1)
        t = jnp.where(cols < crelu, jnp.maximum(t, 0.0), t)
    o_ref[...] = t


def _make_combine(crelu):
    return pl.pallas_call(
        functools.partial(_combine_body, crelu=crelu),
        grid=(N_PAD // BM,),
        in_specs=[
            pl.BlockSpec((NC, BM, C), lambda i: (0, i, 0)),
            pl.BlockSpec((BM, C), lambda i: (i, 0)),
            pl.BlockSpec((BM, 1), lambda i: (i, 0)),
            pl.BlockSpec((1, C), lambda i: (0, 0)),
        ],
        out_specs=pl.BlockSpec((BM, C), lambda i: (i, 0)),
        out_shape=jax.ShapeDtypeStruct((N_PAD, C), jnp.float32),
    )


_combine_relu = _make_combine(C)
_combine_plain = _make_combine(0)
_combine_relu64 = _make_combine(64)

BS = 512


def _bigmm_body(a_ref, b_ref, o_ref):
    o_ref[...] = lax.dot_general(
        a_ref[...], b_ref[...], (((1,), (1,)), ((), ())),
        preferred_element_type=jnp.float32)


_bigmm = pl.pallas_call(
    _bigmm_body,
    grid=(pl.cdiv(N, BS), pl.cdiv(N, BS)),
    in_specs=[
        pl.BlockSpec((BS, H), lambda i, j: (i, 0)),
        pl.BlockSpec((BS, H), lambda i, j: (j, 0)),
    ],
    out_specs=pl.BlockSpec((BS, BS), lambda i, j: (i, j)),
    out_shape=jax.ShapeDtypeStruct((N, N), jnp.float32),
)


def _pad_w(w):
    out = jnp.zeros((C, C), jnp.float32)
    return out.at[: w.shape[0], : w.shape[1]].set(w)


def _pad_b(b):
    out = jnp.zeros((C,), jnp.float32)
    return out.at[: b.shape[0]].set(b).reshape(1, C)


def kernel(x, edge_index, W_e1, b_e1, W_e2, b_e2, W_a1, b_a1, W_a2, b_a2,
           W_s, b_s):
    src = edge_index[0].astype(jnp.int32)
    dst = edge_index[1].astype(jnp.int32)
    pad = jnp.full((E_PAD - E,), N_PAD - 1, jnp.int32)
    srcp = jnp.concatenate([src, pad]).reshape(NW, CH, K)
    dstp = jnp.concatenate([dst, pad]).reshape(NW, CH, K)

    xp = jnp.zeros((N_PAD, C), jnp.float32).at[:N].set(x)
    ones128 = jnp.ones((K, C), jnp.float32)
    z128 = jnp.zeros((RPT, C), jnp.float32)

    deg2 = _deg(dstp, ones128, z128)
    dinv = _finish_deg(deg2)

    u1 = _mm_scale(xp, _pad_w(W_e1), dinv)
    seg1 = _segsum(u1, srcp, dstp, z128)
    h = _combine_relu(seg1, u1, dinv, _pad_b(b_e1))

    u2 = _mm_scale(h, _pad_w(W_e2), dinv)
    seg2 = _segsum(u2, srcp, dstp, z128)
    emb_p = _combine_plain(seg2, u2, dinv, _pad_b(b_e2))

    W3 = _pad_w(jnp.concatenate([W_a1, W_s], axis=1))
    b3 = _pad_b(jnp.concatenate([b_a1, b_s]))
    u3 = _mm_scale(emb_p, W3, dinv)
    seg3 = _segsum(u3, srcp, dstp, z128)
    t3 = _combine_relu64(seg3, u3, dinv, b3)
    hs = t3[:N, H:]

    W4 = jnp.zeros((C, C), jnp.float32).at[:H].set(W_a2)
    u4 = _mm_scale(t3, W4, dinv)
    seg4 = _segsum(u4, srcp, dstp, z128)
    x_p = _combine_plain(seg4, u4, dinv, b_a2.reshape(1, C))

    s_ = _bigmm(hs, hs)
    return (x_p[:N], s_, emb_p[:N, :H])


# speedup vs baseline: 7.4908x; 1.1088x over previous
import functools

import jax
import jax.numpy as jnp
from jax import lax
from jax.experimental import pallas as pl
from jax.experimental.pallas import tpu as pltpu
from jax.experimental.pallas import tpu_sc as plsc

N = 10000
E = 320000
D_IN = 128
H = 64
C = 128

NC = 2
NS = 16
K = 128
IB = 8
CH0 = 112
CH1 = 48
CHT = CH0 + CH1
NB0 = CH0 // IB
NB1 = CH1 // IB
E_PAD = K * CHT * NS
N_PAD = 10240
RPT = N_PAD // NS

_mesh = plsc.VectorSubcoreMesh(core_axis_name="c", subcore_axis_name="s")


@functools.partial(
    pl.kernel,
    out_type=jax.ShapeDtypeStruct((NC, N_PAD, C), jnp.float32),
    mesh=_mesh,
    scratch_types=[
        pltpu.VMEM((IB, K), jnp.int32),
        pltpu.VMEM((IB, K), jnp.int32),
        pltpu.VMEM((K, C), jnp.float32),
        pltpu.VMEM((K, C), jnp.float32),
        pltpu.VMEM_SHARED((N_PAD, C), jnp.float32),
        pltpu.SemaphoreType.DMA,
        pltpu.SemaphoreType.DMA,
    ],
)
def _segsum(h_hbm, srcp_hbm, dstp_hbm, zeros_hbm, out_hbm,
            src_v, dst_v, rows0, rows1, acc, sem0, sem1):
    cid = lax.axis_index("c")
    sid = lax.axis_index("s")
    base = cid * CH0
    nbs = jnp.where(cid == 0, NB0, NB1)
    r0 = sid * RPT
    pltpu.sync_copy(zeros_hbm, acc.at[pl.ds(r0, RPT)])
    plsc.subcore_barrier()

    bufs = ((rows0, sem0), (rows1, sem1))

    def block(nb, carry):
        c0 = base + nb * IB
        pltpu.sync_copy(srcp_hbm.at[sid, pl.ds(c0, IB)], src_v)
        pltpu.sync_copy(dstp_hbm.at[sid, pl.ds(c0, IB)], dst_v)
        pltpu.async_copy(h_hbm.at[src_v.at[0]], rows0, sem0)
        pltpu.async_copy(h_hbm.at[src_v.at[1]], rows1, sem1)
        for j in range(IB):
            rows, sem = bufs[j % 2]
            pltpu.make_async_copy(h_hbm.at[src_v.at[j]], rows, sem).wait()
            pltpu.sync_copy(rows, acc.at[dst_v.at[j]], add=True)
            if j + 2 < IB:
                pltpu.async_copy(h_hbm.at[src_v.at[j + 2]], rows, sem)
        return carry

    lax.fori_loop(0, nbs, block, 0)
    plsc.subcore_barrier()
    pltpu.sync_copy(acc.at[pl.ds(r0, RPT)],
                    out_hbm.at[cid, pl.ds(r0, RPT)])


@functools.partial(
    pl.kernel,
    out_type=jax.ShapeDtypeStruct((NC, N_PAD, C), jnp.float32),
    mesh=_mesh,
    scratch_types=[
        pltpu.VMEM((IB, K), jnp.int32),
        pltpu.VMEM((K, C), jnp.float32),
        pltpu.VMEM_SHARED((N_PAD, C), jnp.float32),
    ],
)
def _deg(dstp_hbm, ones_hbm, zeros_hbm, out_hbm, dst_v, ones_v, acc):
    cid = lax.axis_index("c")
    sid = lax.axis_index("s")
    base = cid * CH0
    nbs = jnp.where(cid == 0, NB0, NB1)
    pltpu.sync_copy(ones_hbm, ones_v)
    r0 = sid * RPT
    pltpu.sync_copy(zeros_hbm, acc.at[pl.ds(r0, RPT)])
    plsc.subcore_barrier()

    def block(nb, carry):
        pltpu.sync_copy(dstp_hbm.at[sid, pl.ds(base + nb * IB, IB)], dst_v)
        for j in range(IB):
            pltpu.sync_copy(ones_v, acc.at[dst_v.at[j]], add=True)
        return carry

    lax.fori_loop(0, nbs, block, 0)
    plsc.subcore_barrier()
    pltpu.sync_copy(acc.at[pl.ds(r0, RPT)], out_hbm.at[cid, pl.ds(r0, RPT)])


BM = 2048


def _finish_deg_body(deg_ref, o_ref):
    d = deg_ref[...]
    o_ref[...] = lax.rsqrt(d[0, :, 0:1] + d[1, :, 0:1] + 1.0)


_finish_deg = pl.pallas_call(
    _finish_deg_body,
    grid=(N_PAD // BM,),
    in_specs=[pl.BlockSpec((NC, BM, C), lambda i: (0, i, 0))],
    out_specs=pl.BlockSpec((BM, 1), lambda i: (i, 0)),
    out_shape=jax.ShapeDtypeStruct((N_PAD, 1), jnp.float32),
)


def _mm_scale_body(p_ref, w_ref, dv_ref, o_ref):
    o_ref[...] = dv_ref[...] * jnp.dot(
        p_ref[...], w_ref[...], preferred_element_type=jnp.float32)


_mm_scale = pl.pallas_call(
    _mm_scale_body,
    grid=(N_PAD // BM,),
    in_specs=[
        pl.BlockSpec((BM, C), lambda i: (i, 0)),
        pl.BlockSpec((C, C), lambda i: (0, 0)),
        pl.BlockSpec((BM, 1), lambda i: (i, 0)),
    ],
    out_specs=pl.BlockSpec((BM, C), lambda i: (i, 0)),
    out_shape=jax.ShapeDtypeStruct((N_PAD, C), jnp.float32),
)


def _combine_body(seg_ref, u_ref, dv_ref, b_ref, o_ref, *, crelu):
    t = dv_ref[...] * (seg_ref[0] + seg_ref[1] + u_ref[...]) + b_ref[...]
    if crelu == t.shape[1]:
        t = jnp.maximum(t, 0.0)
    elif crelu > 0:
        cols = lax.broadcasted_iota(jnp.int32, t.shape, 1)
        t = jnp.where(cols < crelu, jnp.maximum(t, 0.0), t)
    o_ref[...] = t


def _make_combine(crelu):
    return pl.pallas_call(
        functools.partial(_combine_body, crelu=crelu),
        grid=(N_PAD // BM,),
        in_specs=[
            pl.BlockSpec((NC, BM, C), lambda i: (0, i, 0)),
            pl.BlockSpec((BM, C), lambda i: (i, 0)),
            pl.BlockSpec((BM, 1), lambda i: (i, 0)),
            pl.BlockSpec((1, C), lambda i: (0, 0)),
        ],
        out_specs=pl.BlockSpec((BM, C), lambda i: (i, 0)),
        out_shape=jax.ShapeDtypeStruct((N_PAD, C), jnp.float32),
    )


_combine_relu = _make_combine(C)
_combine_plain = _make_combine(0)
_combine_relu64 = _make_combine(64)

BS = 512


def _bigmm_body(a_ref, b_ref, o_ref):
    o_ref[...] = lax.dot_general(
        a_ref[...], b_ref[...], (((1,), (1,)), ((), ())),
        preferred_element_type=jnp.float32)


_bigmm = pl.pallas_call(
    _bigmm_body,
    grid=(pl.cdiv(N, BS), pl.cdiv(N, BS)),
    in_specs=[
        pl.BlockSpec((BS, H), lambda i, j: (i, 0)),
        pl.BlockSpec((BS, H), lambda i, j: (j, 0)),
    ],
    out_specs=pl.BlockSpec((BS, BS), lambda i, j: (i, j)),
    out_shape=jax.ShapeDtypeStruct((N, N), jnp.float32),
)


def _pad_w(w):
    out = jnp.zeros((C, C), jnp.float32)
    return out.at[: w.shape[0], : w.shape[1]].set(w)


def _pad_b(b):
    out = jnp.zeros((C,), jnp.float32)
    return out.at[: b.shape[0]].set(b).reshape(1, C)


def kernel(x, edge_index, W_e1, b_e1, W_e2, b_e2, W_a1, b_a1, W_a2, b_a2,
           W_s, b_s):
    src = edge_index[0].astype(jnp.int32)
    dst = edge_index[1].astype(jnp.int32)
    pad = jnp.full((E_PAD - E,), N_PAD - 1, jnp.int32)
    srcp = jnp.concatenate([src, pad]).reshape(NS, CHT, K)
    dstp = jnp.concatenate([dst, pad]).reshape(NS, CHT, K)

    xp = jnp.zeros((N_PAD, C), jnp.float32).at[:N].set(x)
    ones128 = jnp.ones((K, C), jnp.float32)
    z128 = jnp.zeros((RPT, C), jnp.float32)

    deg2 = _deg(dstp, ones128, z128)
    dinv = _finish_deg(deg2)

    u1 = _mm_scale(xp, _pad_w(W_e1), dinv)
    seg1 = _segsum(u1, srcp, dstp, z128)
    h = _combine_relu(seg1, u1, dinv, _pad_b(b_e1))

    u2 = _mm_scale(h, _pad_w(W_e2), dinv)
    seg2 = _segsum(u2, srcp, dstp, z128)
    emb_p = _combine_plain(seg2, u2, dinv, _pad_b(b_e2))

    W3 = _pad_w(jnp.concatenate([W_a1, W_s], axis=1))
    b3 = _pad_b(jnp.concatenate([b_a1, b_s]))
    u3 = _mm_scale(emb_p, W3, dinv)
    seg3 = _segsum(u3, srcp, dstp, z128)
    t3 = _combine_relu64(seg3, u3, dinv, b3)
    hs = t3[:N, H:]

    W4 = jnp.zeros((C, C), jnp.float32).at[:H].set(W_a2)
    u4 = _mm_scale(t3, W4, dinv)
    seg4 = _segsum(u4, srcp, dstp, z128)
    x_p = _combine_plain(seg4, u4, dinv, b_a2.reshape(1, C))

    s_ = _bigmm(hs, hs)
    return (x_p[:N], s_, emb_p[:N, :H])
